# baseline (speedup 1.0000x reference)
import functools

import jax
import jax.numpy as jnp
from jax import lax
from jax.experimental import pallas as pl
from jax.experimental.pallas import tpu as pltpu
from jax.experimental.pallas import tpu_sc as plsc

N = 10000
E = 320000
D_IN = 128
D_HID = 128
D_LAT = 64

NC = 2
NS = 16
LANES = 16

NP = 10240
ROWS_PER_TILE = NP // NS
CHUNK = 128
EPT = 20480
NCHUNK = EPT // CHUNK
E_PAD = EPT * NS
DEG_EPT = E_PAD // (NC * NS)
DEG_ROWS = DEG_EPT // CHUNK
H = 64


def _deg_body(dst_hbm, out_hbm, dst_v, acc_v, ones_v, sem):
    c = lax.axis_index("c")
    s = lax.axis_index("s")
    wid = s * NC + c
    pltpu.sync_copy(dst_hbm.at[wid], dst_v)
    zero16 = jnp.zeros((LANES,), jnp.float32)

    def zero_body(i, _):
        acc_v[pl.ds(i * LANES, LANES)] = zero16
        return 0

    lax.fori_loop(0, NP // LANES, zero_body, 0)
    ones_v[...] = jnp.ones((LANES,), jnp.float32)
    one = ones_v[...]

    def row_body(k, _):
        for j in range(CHUNK // LANES):
            idx = dst_v[k, pl.ds(j * LANES, LANES)]
            plsc.addupdate_scatter(acc_v, (idx,), one)
        return 0

    lax.fori_loop(0, DEG_ROWS, row_body, 0)
    pltpu.sync_copy(acc_v, out_hbm.at[wid])


def _make_deg_kernel():
    mesh = plsc.VectorSubcoreMesh(core_axis_name="c", subcore_axis_name="s")
    return pl.kernel(
        _deg_body,
        out_type=jax.ShapeDtypeStruct((NC * NS, NP), jnp.float32),
        mesh=mesh,
        compiler_params=pltpu.CompilerParams(needs_layout_passes=False),
        scratch_types=[
            pltpu.VMEM((DEG_ROWS, CHUNK), jnp.int32),
            pltpu.VMEM((NP,), jnp.float32),
            pltpu.VMEM((LANES,), jnp.float32),
            pltpu.SemaphoreType.DMA,
        ],
    )


NBUF = 4
GC = 128
NGROUP = EPT // GC
SG = 8
NSUPER = NGROUP // SG
ISLOT = 4


def _idx_row(src_sv, dst_sv, p, g):
    return src_sv.at[(p + g // SG) % ISLOT, g % SG], \
           dst_sv.at[(p + g // SG) % ISLOT, g % SG]


def _prop_body(u_hbm, src_hbm, dst_hbm, zeros_hbm, y_hbm,
               src_sv, dst_sv, gbufs, u_sh, acc_sh, sem, sem2, sem_i):
    c = lax.axis_index("c")
    s = lax.axis_index("s")
    rows = pl.ds(s * ROWS_PER_TILE, ROWS_PER_TILE)
    pltpu.sync_copy(zeros_hbm, acc_sh.at[rows])
    pltpu.sync_copy(u_hbm.at[c].at[rows], u_sh.at[rows])

    src_s = src_hbm.at[s]
    dst_s = dst_hbm.at[s]

    def idx_load(sp, slot):
        pltpu.async_copy(src_s.at[sp], src_sv.at[slot], sem_i)
        pltpu.async_copy(dst_s.at[sp], dst_sv.at[slot], sem_i)

    def idx_wait(slot):
        pltpu.make_async_copy(src_s.at[0], src_sv.at[slot], sem_i).wait()
        pltpu.make_async_copy(dst_s.at[0], dst_sv.at[slot], sem_i).wait()

    idx_load(0, 0)
    idx_load(1, 1)
    idx_wait(0)
    plsc.subcore_barrier()

    for b in range(NBUF):
        pltpu.async_copy(u_sh.at[src_sv.at[0, b]], gbufs.at[b], sem)

    @pl.loop(0, NSUPER, step=ISLOT)
    def _supers(sb):
        for p in range(ISLOT):
            sp = sb + p
            @pl.when(sp + 1 < NSUPER)
            def _():
                idx_wait((p + 1) % ISLOT)

            @pl.when(sp + 2 < NSUPER)
            def _():
                idx_load(sp + 2, (p + 2) % ISLOT)

            for g in range(SG):
                k = sp * SG + g
                b = g % NBUF
                s_row, d_row = _idx_row(src_sv, dst_sv, p, g)
                pltpu.make_async_copy(u_sh.at[s_row], gbufs.at[b], sem).wait()
                pltpu.async_copy(gbufs.at[b], acc_sh.at[d_row], sem2, add=True)
                bn = (g + 1) % NBUF
                _, d_old = _idx_row(src_sv, dst_sv, p, g + 1 - NBUF)
                s_new, _ = _idx_row(src_sv, dst_sv, p, g + 1 + NBUF - 1)
                j = k - NBUF + 1

                @pl.when(jnp.logical_and(j >= 0, j + NBUF < NGROUP))
                def _():
                    pltpu.make_async_copy(gbufs.at[bn], acc_sh.at[d_old],
                                          sem2).wait()
                    pltpu.async_copy(u_sh.at[s_new], gbufs.at[bn], sem)

    for b in range(NBUF):
        pltpu.make_async_copy(gbufs.at[b], acc_sh.at[dst_sv.at[0, 0]],
                              sem2).wait()

    plsc.subcore_barrier()
    pltpu.sync_copy(acc_sh.at[rows], y_hbm.at[c].at[rows])


def _make_prop_kernel():
    mesh = plsc.VectorSubcoreMesh(core_axis_name="c", subcore_axis_name="s")
    return pl.kernel(
        _prop_body,
        out_type=jax.ShapeDtypeStruct((NC, NP, H), jnp.float32),
        mesh=mesh,
        compiler_params=pltpu.CompilerParams(use_tc_tiling_on_sc=False),
        scratch_types=[
            pltpu.VMEM((ISLOT, SG, GC), jnp.int32),
            pltpu.VMEM((ISLOT, SG, GC), jnp.int32),
            pltpu.VMEM((NBUF, GC, H), jnp.float32),
            pltpu.VMEM_SHARED((NP, H), jnp.float32),
            pltpu.VMEM_SHARED((NP, H), jnp.float32),
            pltpu.SemaphoreType.DMA,
            pltpu.SemaphoreType.DMA,
            pltpu.SemaphoreType.DMA,
        ],
    )


TCR = 2048


def _dinv_from_partials(degp):
    deg = jnp.sum(degp, axis=0) + 1.0
    return lax.rsqrt(jnp.maximum(deg, 1.0))


def _stage_a_body(x_ref, degp_ref, w1_ref, u_ref):
    dinv = _dinv_from_partials(degp_ref[...])
    u = jnp.dot(x_ref[...], w1_ref[...], preferred_element_type=jnp.float32)
    u = u * dinv[:, None]
    u_ref[0] = u[:, :H]
    u_ref[1] = u[:, H:]


def _stage_b_body(y_ref, uin_ref, degp_ref, w23_ref, u_ref):
    dinv = _dinv_from_partials(degp_ref[...])
    tot = y_ref[...] + uin_ref[...]
    h1 = jnp.concatenate([tot[0], tot[1]], axis=-1) * dinv[:, None]
    h1 = jnp.maximum(h1, 0.0)
    u2 = jnp.dot(h1, w23_ref[...], preferred_element_type=jnp.float32)
    u2 = u2 * dinv[:, None]
    u_ref[0] = u2[:, :H]
    u_ref[1] = u2[:, H:]


def _stage_c_body(y_ref, u_ref, degp_ref, mu_ref, lv_ref):
    dinv = _dinv_from_partials(degp_ref[...])
    tot = y_ref[...] + u_ref[...]
    mu_ref[...] = tot[0] * dinv[:, None]
    lv_ref[...] = tot[1] * dinv[:, None]


def _row_spec(feat):
    return pl.BlockSpec((TCR, feat), lambda i: (i, 0))


def _split_spec():
    return pl.BlockSpec((NC, TCR, H), lambda i: (0, i, 0))


def _degp_spec():
    return pl.BlockSpec((NC * NS, TCR), lambda i: (0, i))


def _full_spec(shape):
    return pl.BlockSpec(shape, lambda i: tuple(0 for _ in shape))


_GRID = (NP // TCR,)

_stage_a = pl.pallas_call(
    _stage_a_body,
    grid=_GRID,
    in_specs=[_row_spec(D_IN), _degp_spec(), _full_spec((D_IN, D_HID))],
    out_specs=[_split_spec()],
    out_shape=[jax.ShapeDtypeStruct((NC, NP, H), jnp.float32)],
)

_stage_b = pl.pallas_call(
    _stage_b_body,
    grid=_GRID,
    in_specs=[_split_spec(), _split_spec(), _degp_spec(),
              _full_spec((D_HID, 2 * D_LAT))],
    out_specs=[_split_spec()],
    out_shape=[jax.ShapeDtypeStruct((NC, NP, H), jnp.float32)],
)

_stage_c = pl.pallas_call(
    _stage_c_body,
    grid=_GRID,
    in_specs=[_split_spec(), _split_spec(), _degp_spec()],
    out_specs=[_row_spec(D_LAT), _row_spec(D_LAT)],
    out_shape=[jax.ShapeDtypeStruct((NP, D_LAT), jnp.float32)] * 2,
)

_deg_kernel = _make_deg_kernel()
_prop_kernel = _make_prop_kernel()


def kernel(x, edge_index, W1, W2, W3):
    src = edge_index[0]
    dst = edge_index[1]
    pad_e = E_PAD - E
    pad_idx = jnp.full((pad_e,), N, dtype=jnp.int32)
    srcr = jnp.concatenate([src, pad_idx]).reshape(NS, NSUPER, SG, GC)
    dstr = jnp.concatenate([dst, pad_idx]).reshape(NS, NSUPER, SG, GC)
    dst_deg = jnp.concatenate([dst, pad_idx]).reshape(NC * NS, DEG_ROWS, CHUNK)

    xp = jnp.pad(x, ((0, NP - N), (0, 0)))
    w23 = jnp.concatenate([W2, W3], axis=1)
    zeros_tile = jnp.zeros((ROWS_PER_TILE, H), jnp.float32)

    degp = _deg_kernel(dst_deg)

    (u1,) = _stage_a(xp, degp, W1)
    y1 = _prop_kernel(u1, srcr, dstr, zeros_tile)

    (u2,) = _stage_b(y1, u1, degp, w23)
    y2 = _prop_kernel(u2, srcr, dstr, zeros_tile)

    mu, logvar = _stage_c(y2, u2, degp)
    return (mu[:N], logvar[:N])


# speedup vs baseline: 1.0029x; 1.0029x over previous
import functools

import jax
import jax.numpy as jnp
from jax import lax
from jax.experimental import pallas as pl
from jax.experimental.pallas import tpu as pltpu
from jax.experimental.pallas import tpu_sc as plsc

N = 10000
E = 320000
D_IN = 128
D_HID = 128
D_LAT = 64

NC = 2
NS = 16
LANES = 16

NP = 10240
ROWS_PER_TILE = NP // NS
CHUNK = 128
EPT = 20480
NCHUNK = EPT // CHUNK
E_PAD = EPT * NS
DEG_EPT = E_PAD // (NC * NS)
DEG_ROWS = DEG_EPT // CHUNK
H = 64


def _deg_body(dst_hbm, out_hbm, dst_v, acc_v, ones_v, sem):
    c = lax.axis_index("c")
    s = lax.axis_index("s")
    wid = s * NC + c
    pltpu.sync_copy(dst_hbm.at[wid], dst_v)
    zero16 = jnp.zeros((LANES,), jnp.float32)

    def zero_body(i, _):
        acc_v[pl.ds(i * LANES, LANES)] = zero16
        return 0

    lax.fori_loop(0, NP // LANES, zero_body, 0)
    ones_v[...] = jnp.ones((LANES,), jnp.float32)
    one = ones_v[...]

    def row_body(k, _):
        for j in range(CHUNK // LANES):
            idx = dst_v[k, pl.ds(j * LANES, LANES)]
            plsc.addupdate_scatter(acc_v, (idx,), one)
        return 0

    lax.fori_loop(0, DEG_ROWS, row_body, 0)
    pltpu.sync_copy(acc_v, out_hbm.at[wid])


def _make_deg_kernel():
    mesh = plsc.VectorSubcoreMesh(core_axis_name="c", subcore_axis_name="s")
    return pl.kernel(
        _deg_body,
        out_type=jax.ShapeDtypeStruct((NC * NS, NP), jnp.float32),
        mesh=mesh,
        compiler_params=pltpu.CompilerParams(needs_layout_passes=False),
        scratch_types=[
            pltpu.VMEM((DEG_ROWS, CHUNK), jnp.int32),
            pltpu.VMEM((NP,), jnp.float32),
            pltpu.VMEM((LANES,), jnp.float32),
            pltpu.SemaphoreType.DMA,
        ],
    )


NBUF = 4
GC = 128
NGROUP = EPT // GC
SG = 8
NSUPER = NGROUP // SG
ISLOT = 4


def _idx_row(src_sv, dst_sv, p, g):
    return src_sv.at[(p + g // SG) % ISLOT, g % SG], \
           dst_sv.at[(p + g // SG) % ISLOT, g % SG]


def _prop_body(u_hbm, src_hbm, dst_hbm, zeros_hbm, y_hbm,
               src_sv, dst_sv, gbufs, u_sh, acc_sh, sem, sem2, sem_i):
    c = lax.axis_index("c")
    s = lax.axis_index("s")
    rows = pl.ds(s * ROWS_PER_TILE, ROWS_PER_TILE)
    pltpu.sync_copy(zeros_hbm, acc_sh.at[rows])
    pltpu.sync_copy(u_hbm.at[c].at[rows], u_sh.at[rows])

    src_s = src_hbm.at[s]
    dst_s = dst_hbm.at[s]

    def idx_load(sp, slot):
        pltpu.async_copy(src_s.at[sp], src_sv.at[slot], sem_i)
        pltpu.async_copy(dst_s.at[sp], dst_sv.at[slot], sem_i)

    def idx_wait(slot):
        pltpu.make_async_copy(src_s.at[0], src_sv.at[slot], sem_i).wait()
        pltpu.make_async_copy(dst_s.at[0], dst_sv.at[slot], sem_i).wait()

    idx_load(0, 0)
    idx_load(1, 1)
    idx_wait(0)
    plsc.subcore_barrier()

    for b in range(NBUF):
        pltpu.async_copy(u_sh.at[src_sv.at[0, b]], gbufs.at[b], sem)

    @pl.loop(0, NSUPER, step=ISLOT)
    def _supers(sb):
        for p in range(ISLOT):
            sp = sb + p
            @pl.when(sp + 1 < NSUPER)
            def _():
                idx_wait((p + 1) % ISLOT)

            @pl.when(sp + 2 < NSUPER)
            def _():
                idx_load(sp + 2, (p + 2) % ISLOT)

            for g in range(SG):
                k = sp * SG + g
                b = g % NBUF
                s_row, d_row = _idx_row(src_sv, dst_sv, p, g)
                pltpu.make_async_copy(u_sh.at[s_row], gbufs.at[b], sem).wait()
                pltpu.async_copy(gbufs.at[b], acc_sh.at[d_row], sem2, add=True)
                bn = (g + 1) % NBUF
                _, d_old = _idx_row(src_sv, dst_sv, p, g + 1 - NBUF)
                s_new, _ = _idx_row(src_sv, dst_sv, p, g + 1)
                j = k - NBUF + 1

                @pl.when(jnp.logical_and(j >= 0, j + NBUF < NGROUP))
                def _():
                    pltpu.make_async_copy(gbufs.at[bn], acc_sh.at[d_old],
                                          sem2).wait()
                    pltpu.async_copy(u_sh.at[s_new], gbufs.at[bn], sem)

    for b in range(NBUF):
        pltpu.make_async_copy(gbufs.at[b], acc_sh.at[dst_sv.at[0, 0]],
                              sem2).wait()

    plsc.subcore_barrier()
    pltpu.sync_copy(acc_sh.at[rows], y_hbm.at[c].at[rows])


def _make_prop_kernel():
    mesh = plsc.VectorSubcoreMesh(core_axis_name="c", subcore_axis_name="s")
    return pl.kernel(
        _prop_body,
        out_type=jax.ShapeDtypeStruct((NC, NP, H), jnp.float32),
        mesh=mesh,
        compiler_params=pltpu.CompilerParams(use_tc_tiling_on_sc=False),
        scratch_types=[
            pltpu.VMEM((ISLOT, SG, GC), jnp.int32),
            pltpu.VMEM((ISLOT, SG, GC), jnp.int32),
            pltpu.VMEM((NBUF, GC, H), jnp.float32),
            pltpu.VMEM_SHARED((NP, H), jnp.float32),
            pltpu.VMEM_SHARED((NP, H), jnp.float32),
            pltpu.SemaphoreType.DMA,
            pltpu.SemaphoreType.DMA,
            pltpu.SemaphoreType.DMA,
        ],
    )


TCR = 2048


def _dinv_from_partials(degp):
    deg = jnp.sum(degp, axis=0) + 1.0
    return lax.rsqrt(jnp.maximum(deg, 1.0))


def _stage_a_body(x_ref, degp_ref, w1_ref, u_ref):
    dinv = _dinv_from_partials(degp_ref[...])
    u = jnp.dot(x_ref[...], w1_ref[...], preferred_element_type=jnp.float32)
    u = u * dinv[:, None]
    u_ref[0] = u[:, :H]
    u_ref[1] = u[:, H:]


def _stage_b_body(y_ref, uin_ref, degp_ref, w23_ref, u_ref):
    dinv = _dinv_from_partials(degp_ref[...])
    tot = y_ref[...] + uin_ref[...]
    h1 = jnp.concatenate([tot[0], tot[1]], axis=-1) * dinv[:, None]
    h1 = jnp.maximum(h1, 0.0)
    u2 = jnp.dot(h1, w23_ref[...], preferred_element_type=jnp.float32)
    u2 = u2 * dinv[:, None]
    u_ref[0] = u2[:, :H]
    u_ref[1] = u2[:, H:]


def _stage_c_body(y_ref, u_ref, degp_ref, mu_ref, lv_ref):
    dinv = _dinv_from_partials(degp_ref[...])
    tot = y_ref[...] + u_ref[...]
    mu_ref[...] = tot[0] * dinv[:, None]
    lv_ref[...] = tot[1] * dinv[:, None]


def _row_spec(feat):
    return pl.BlockSpec((TCR, feat), lambda i: (i, 0))


def _split_spec():
    return pl.BlockSpec((NC, TCR, H), lambda i: (0, i, 0))


def _degp_spec():
    return pl.BlockSpec((NC * NS, TCR), lambda i: (0, i))


def _full_spec(shape):
    return pl.BlockSpec(shape, lambda i: tuple(0 for _ in shape))


_GRID = (NP // TCR,)

_stage_a = pl.pallas_call(
    _stage_a_body,
    grid=_GRID,
    in_specs=[_row_spec(D_IN), _degp_spec(), _full_spec((D_IN, D_HID))],
    out_specs=[_split_spec()],
    out_shape=[jax.ShapeDtypeStruct((NC, NP, H), jnp.float32)],
)

_stage_b = pl.pallas_call(
    _stage_b_body,
    grid=_GRID,
    in_specs=[_split_spec(), _split_spec(), _degp_spec(),
              _full_spec((D_HID, 2 * D_LAT))],
    out_specs=[_split_spec()],
    out_shape=[jax.ShapeDtypeStruct((NC, NP, H), jnp.float32)],
)

_stage_c = pl.pallas_call(
    _stage_c_body,
    grid=_GRID,
    in_specs=[_split_spec(), _split_spec(), _degp_spec()],
    out_specs=[_row_spec(D_LAT), _row_spec(D_LAT)],
    out_shape=[jax.ShapeDtypeStruct((NP, D_LAT), jnp.float32)] * 2,
)

_deg_kernel = _make_deg_kernel()
_prop_kernel = _make_prop_kernel()


def kernel(x, edge_index, W1, W2, W3):
    src = edge_index[0]
    dst = edge_index[1]
    pad_e = E_PAD - E
    pad_idx = jnp.full((pad_e,), N, dtype=jnp.int32)
    srcr = jnp.concatenate([src, pad_idx]).reshape(NS, NSUPER, SG, GC)
    dstr = jnp.concatenate([dst, pad_idx]).reshape(NS, NSUPER, SG, GC)
    dst_deg = jnp.concatenate([dst, pad_idx]).reshape(NC * NS, DEG_ROWS, CHUNK)

    xp = jnp.pad(x, ((0, NP - N), (0, 0)))
    w23 = jnp.concatenate([W2, W3], axis=1)
    zeros_tile = jnp.zeros((ROWS_PER_TILE, H), jnp.float32)

    degp = _deg_kernel(dst_deg)

    (u1,) = _stage_a(xp, degp, W1)
    y1 = _prop_kernel(u1, srcr, dstr, zeros_tile)

    (u2,) = _stage_b(y1, u1, degp, w23)
    y2 = _prop_kernel(u2, srcr, dstr, zeros_tile)

    mu, logvar = _stage_c(y2, u2, degp)
    return (mu[:N], logvar[:N])


# speedup vs baseline: 1.0067x; 1.0038x over previous
import functools

import jax
import jax.numpy as jnp
from jax import lax
from jax.experimental import pallas as pl
from jax.experimental.pallas import tpu as pltpu
from jax.experimental.pallas import tpu_sc as plsc

N = 10000
E = 320000
D_IN = 128
D_HID = 128
D_LAT = 64

NC = 2
NS = 16
LANES = 16

NP = 10240
ROWS_PER_TILE = NP // NS
CHUNK = 128
EPT = 20480
NCHUNK = EPT // CHUNK
E_PAD = EPT * NS
DEG_EPT = E_PAD // (NC * NS)
DEG_ROWS = DEG_EPT // CHUNK
H = 64
TCR = 2000
NSTG = N // TCR


SUP_PER_DEG_TILE = 10


def _deg_body(dst_hbm, out_hbm, dst_v, acc_v, ones_v, sem):
    c = lax.axis_index("c")
    s = lax.axis_index("s")
    wid = s * NC + c
    pltpu.sync_copy(dst_hbm.at[s].at[pl.ds(c * SUP_PER_DEG_TILE,
                                           SUP_PER_DEG_TILE)], dst_v)
    zero16 = jnp.zeros((LANES,), jnp.float32)

    def zero_body(i, _):
        acc_v[pl.ds(i * LANES, LANES)] = zero16
        return 0

    lax.fori_loop(0, NP // LANES, zero_body, 0)
    ones_v[...] = jnp.ones((LANES,), jnp.float32)
    one = ones_v[...]

    def row_body(k, _):
        q = k // SG
        r = k % SG
        for j in range(CHUNK // LANES):
            idx = dst_v[q, r, pl.ds(j * LANES, LANES)]
            plsc.addupdate_scatter(acc_v, (idx,), one)
        return 0

    lax.fori_loop(0, SUP_PER_DEG_TILE * SG, row_body, 0)
    for q in range(NSTG):
        pltpu.sync_copy(acc_v.at[pl.ds(q * TCR, TCR)], out_hbm.at[q, wid])


def _make_deg_kernel():
    mesh = plsc.VectorSubcoreMesh(core_axis_name="c", subcore_axis_name="s")
    return pl.kernel(
        _deg_body,
        out_type=jax.ShapeDtypeStruct((NSTG, NC * NS, TCR), jnp.float32),
        mesh=mesh,
        compiler_params=pltpu.CompilerParams(needs_layout_passes=False,
                                             use_tc_tiling_on_sc=False),
        scratch_types=[
            pltpu.VMEM((SUP_PER_DEG_TILE, SG, CHUNK), jnp.int32),
            pltpu.VMEM((NP,), jnp.float32),
            pltpu.VMEM((LANES,), jnp.float32),
            pltpu.SemaphoreType.DMA,
        ],
    )


NBUF = 4
GC = 128
NGROUP = EPT // GC
SG = 8
NSUPER = NGROUP // SG
ISLOT = 4


def _idx_row(src_sv, dst_sv, p, g):
    return src_sv.at[(p + g // SG) % ISLOT, g % SG], \
           dst_sv.at[(p + g // SG) % ISLOT, g % SG]


def _prop_body(u_hbm, src_hbm, dst_hbm, zeros_hbm, y_hbm,
               src_sv, dst_sv, gbufs, u_sh, acc_sh, sem, sem2, sem_i):
    c = lax.axis_index("c")
    s = lax.axis_index("s")
    rows = pl.ds(s * ROWS_PER_TILE, ROWS_PER_TILE)
    pltpu.sync_copy(zeros_hbm, acc_sh.at[rows])

    @pl.when(s < NS - 1)
    def _():
        pltpu.sync_copy(u_hbm.at[c].at[rows], u_sh.at[rows])

    @pl.when(s == NS - 1)
    def _():
        tail = pl.ds((NS - 1) * ROWS_PER_TILE, N - (NS - 1) * ROWS_PER_TILE)
        pltpu.sync_copy(u_hbm.at[c].at[tail], u_sh.at[tail])

    src_s = src_hbm.at[s]
    dst_s = dst_hbm.at[s]

    def idx_load(sp, slot):
        pltpu.async_copy(src_s.at[sp], src_sv.at[slot], sem_i)
        pltpu.async_copy(dst_s.at[sp], dst_sv.at[slot], sem_i)

    def idx_wait(slot):
        pltpu.make_async_copy(src_s.at[0], src_sv.at[slot], sem_i).wait()
        pltpu.make_async_copy(dst_s.at[0], dst_sv.at[slot], sem_i).wait()

    idx_load(0, 0)
    idx_load(1, 1)
    idx_wait(0)
    plsc.subcore_barrier()

    for b in range(NBUF):
        pltpu.async_copy(u_sh.at[src_sv.at[0, b]], gbufs.at[b], sem)

    @pl.loop(0, NSUPER, step=ISLOT)
    def _supers(sb):
        for p in range(ISLOT):
            sp = sb + p
            @pl.when(sp + 1 < NSUPER)
            def _():
                idx_wait((p + 1) % ISLOT)

            @pl.when(sp + 2 < NSUPER)
            def _():
                idx_load(sp + 2, (p + 2) % ISLOT)

            for g in range(SG):
                k = sp * SG + g
                b = g % NBUF
                s_row, d_row = _idx_row(src_sv, dst_sv, p, g)
                pltpu.make_async_copy(u_sh.at[s_row], gbufs.at[b], sem).wait()
                pltpu.async_copy(gbufs.at[b], acc_sh.at[d_row], sem2, add=True)
                bn = (g + 1) % NBUF
                _, d_old = _idx_row(src_sv, dst_sv, p, g + 1 - NBUF)
                s_new, _ = _idx_row(src_sv, dst_sv, p, g + 1)
                j = k - NBUF + 1

                @pl.when(jnp.logical_and(j >= 0, j + NBUF < NGROUP))
                def _():
                    pltpu.make_async_copy(gbufs.at[bn], acc_sh.at[d_old],
                                          sem2).wait()
                    pltpu.async_copy(u_sh.at[s_new], gbufs.at[bn], sem)

    for b in range(NBUF):
        pltpu.make_async_copy(gbufs.at[b], acc_sh.at[dst_sv.at[0, 0]],
                              sem2).wait()

    plsc.subcore_barrier()
    pltpu.sync_copy(acc_sh.at[rows], y_hbm.at[c].at[rows])


def _make_prop_kernel():
    mesh = plsc.VectorSubcoreMesh(core_axis_name="c", subcore_axis_name="s")
    return pl.kernel(
        _prop_body,
        out_type=jax.ShapeDtypeStruct((NC, NP, H), jnp.float32),
        mesh=mesh,
        compiler_params=pltpu.CompilerParams(use_tc_tiling_on_sc=False),
        scratch_types=[
            pltpu.VMEM((ISLOT, SG, GC), jnp.int32),
            pltpu.VMEM((ISLOT, SG, GC), jnp.int32),
            pltpu.VMEM((NBUF, GC, H), jnp.float32),
            pltpu.VMEM_SHARED((NP, H), jnp.float32),
            pltpu.VMEM_SHARED((NP, H), jnp.float32),
            pltpu.SemaphoreType.DMA,
            pltpu.SemaphoreType.DMA,
            pltpu.SemaphoreType.DMA,
        ],
    )


def _dinv_from_partials(degp_ref):
    deg = jnp.sum(degp_ref[0], axis=0) + 1.0
    return lax.rsqrt(jnp.maximum(deg, 1.0))


def _stage_a_body(x_ref, degp_ref, w1_ref, u_ref):
    dinv = _dinv_from_partials(degp_ref)
    u = jnp.dot(x_ref[...], w1_ref[...], preferred_element_type=jnp.float32)
    u = u * dinv[:, None]
    u_ref[0] = u[:, :H]
    u_ref[1] = u[:, H:]


def _stage_b_body(y_ref, uin_ref, degp_ref, w23_ref, u_ref):
    dinv = _dinv_from_partials(degp_ref)
    tot = y_ref[...] + uin_ref[...]
    h1 = jnp.concatenate([tot[0], tot[1]], axis=-1) * dinv[:, None]
    h1 = jnp.maximum(h1, 0.0)
    u2 = jnp.dot(h1, w23_ref[...], preferred_element_type=jnp.float32)
    u2 = u2 * dinv[:, None]
    u_ref[0] = u2[:, :H]
    u_ref[1] = u2[:, H:]


def _stage_c_body(y_ref, u_ref, degp_ref, mu_ref, lv_ref):
    dinv = _dinv_from_partials(degp_ref)
    tot = y_ref[...] + u_ref[...]
    mu_ref[...] = tot[0] * dinv[:, None]
    lv_ref[...] = tot[1] * dinv[:, None]


def _row_spec(feat):
    return pl.BlockSpec((TCR, feat), lambda i: (i, 0))


def _split_spec():
    return pl.BlockSpec((NC, TCR, H), lambda i: (0, i, 0))


def _degp_spec():
    return pl.BlockSpec((1, NC * NS, TCR), lambda i: (i, 0, 0))


def _full_spec(shape):
    return pl.BlockSpec(shape, lambda i: tuple(0 for _ in shape))


_GRID = (N // TCR,)

_stage_a = pl.pallas_call(
    _stage_a_body,
    grid=_GRID,
    in_specs=[_row_spec(D_IN), _degp_spec(), _full_spec((D_IN, D_HID))],
    out_specs=[_split_spec()],
    out_shape=[jax.ShapeDtypeStruct((NC, N, H), jnp.float32)],
)

_stage_b = pl.pallas_call(
    _stage_b_body,
    grid=_GRID,
    in_specs=[_split_spec(), _split_spec(), _degp_spec(),
              _full_spec((D_HID, 2 * D_LAT))],
    out_specs=[_split_spec()],
    out_shape=[jax.ShapeDtypeStruct((NC, N, H), jnp.float32)],
)

_stage_c = pl.pallas_call(
    _stage_c_body,
    grid=_GRID,
    in_specs=[_split_spec(), _split_spec(), _degp_spec()],
    out_specs=[_row_spec(D_LAT), _row_spec(D_LAT)],
    out_shape=[jax.ShapeDtypeStruct((N, D_LAT), jnp.float32)] * 2,
)

_deg_kernel = _make_deg_kernel()
_prop_kernel = _make_prop_kernel()


def kernel(x, edge_index, W1, W2, W3):
    src = edge_index[0]
    dst = edge_index[1]
    pad_e = E_PAD - E
    pad_idx = jnp.full((pad_e,), N, dtype=jnp.int32)
    srcr = jnp.concatenate([src, pad_idx]).reshape(NS, NSUPER, SG, GC)
    dstr = jnp.concatenate([dst, pad_idx]).reshape(NS, NSUPER, SG, GC)

    w23 = jnp.concatenate([W2, W3], axis=1)
    zeros_tile = jnp.zeros((ROWS_PER_TILE, H), jnp.float32)

    degp = _deg_kernel(dstr)

    (u1,) = _stage_a(x, degp, W1)
    y1 = _prop_kernel(u1, srcr, dstr, zeros_tile)

    (u2,) = _stage_b(y1, u1, degp, w23)
    y2 = _prop_kernel(u2, srcr, dstr, zeros_tile)

    mu, logvar = _stage_c(y2, u2, degp)
    return (mu, logvar)
